# Initial kernel scaffold; baseline (speedup 1.0000x reference)
import jax
import jax.numpy as jnp
from jax.experimental import pallas as pl


def kernel(x, adj_values, edge_index, W, b):
    raise NotImplementedError("write your pallas kernel here")


# speedup vs baseline: 2.5682x; 2.5682x over previous
import functools

import jax
import jax.numpy as jnp
from jax import lax
from jax.experimental import pallas as pl
from jax.experimental.pallas import tpu as pltpu
from jax.experimental.pallas import tpu_sc as plsc

N = 10000
D = 128
E = 320000

NUM_CORES = 2
NUM_SUBCORES = 16
NUM_TILES = NUM_CORES * NUM_SUBCORES

CHUNK = 128
CHUNKS_PER_TILE = 80
EDGES_PER_TILE = CHUNK * CHUNKS_PER_TILE
E_PAD = EDGES_PER_TILE * NUM_TILES

N_PAD = 10240
ROWS_PER_SUBCORE = N_PAD // NUM_SUBCORES
ZERO_ROWS = 128

_SELU_ALPHA = 1.6732632423543772
_SELU_SCALE = 1.0507009873554805


def _sc_aggregate(x, src, dst, adj):
    mesh = plsc.VectorSubcoreMesh(core_axis_name="c", subcore_axis_name="s")

    @functools.partial(
        pl.kernel,
        mesh=mesh,
        out_type=jax.ShapeDtypeStruct((NUM_CORES * N_PAD, D), jnp.float32),
        scratch_types=[
            pltpu.VMEM((CHUNK,), jnp.int32),
            pltpu.VMEM((CHUNK,), jnp.int32),
            pltpu.VMEM((CHUNK,), jnp.float32),
            pltpu.VMEM((CHUNK, D), jnp.float32),
            pltpu.VMEM_SHARED((N_PAD, D), jnp.float32),
            pltpu.SemaphoreType.DMA,
        ],
    )
    def agg(x_hbm, src_hbm, dst_hbm, adj_hbm, out_hbm,
            srcv, dstv, adjv, rows, acc, sem):
        c = lax.axis_index("c")
        s = lax.axis_index("s")
        wid = c * NUM_SUBCORES + s

        def zero_row(r, carry):
            for g in range(D // 16):
                rows[r, pl.ds(g * 16, 16)] = jnp.zeros((16,), jnp.float32)
            return carry

        lax.fori_loop(0, ZERO_ROWS, zero_row, 0)
        for k in range(ROWS_PER_SUBCORE // ZERO_ROWS):
            pltpu.sync_copy(
                rows.at[pl.ds(0, ZERO_ROWS)],
                acc.at[pl.ds(s * ROWS_PER_SUBCORE + k * ZERO_ROWS, ZERO_ROWS)])
        plsc.subcore_barrier()

        ebase = wid * EDGES_PER_TILE

        def chunk_body(i, carry):
            base = ebase + i * CHUNK
            pltpu.sync_copy(src_hbm.at[pl.ds(base, CHUNK)], srcv)
            pltpu.sync_copy(dst_hbm.at[pl.ds(base, CHUNK)], dstv)
            pltpu.sync_copy(adj_hbm.at[pl.ds(base, CHUNK)], adjv)
            pltpu.async_copy(x_hbm.at[srcv], rows, sem).wait()

            def scale_group(g16, c2):
                r0 = g16 * 16
                avec = adjv[pl.ds(r0, 16)]
                for j in range(16):
                    a = avec[j]
                    for g in range(D // 16):
                        rows[r0 + j, pl.ds(g * 16, 16)] = (
                            rows[r0 + j, pl.ds(g * 16, 16)] * a)
                return c2

            lax.fori_loop(0, CHUNK // 16, scale_group, 0)
            pltpu.sync_copy(rows, acc.at[dstv], add=True)
            return carry

        lax.fori_loop(0, CHUNKS_PER_TILE, chunk_body, 0)
        plsc.subcore_barrier()

        for k in range(ROWS_PER_SUBCORE // ZERO_ROWS):
            r0 = s * ROWS_PER_SUBCORE + k * ZERO_ROWS
            pltpu.sync_copy(acc.at[pl.ds(r0, ZERO_ROWS)],
                            rows.at[pl.ds(0, ZERO_ROWS)])
            pltpu.sync_copy(rows.at[pl.ds(0, ZERO_ROWS)],
                            out_hbm.at[pl.ds(c * N_PAD + r0, ZERO_ROWS)])

    return agg(x, src, dst, adj)


def _finalize_body(p0_ref, p1_ref, w_ref, b_ref, o_ref):
    acc = p0_ref[...] + p1_ref[...]
    h = jnp.dot(acc, w_ref[...], preferred_element_type=jnp.float32)
    neg = _SELU_ALPHA * (jnp.exp(h) - 1.0)
    o_ref[...] = _SELU_SCALE * jnp.where(h > 0, h, neg) + b_ref[...]


def _tc_finalize(p0, p1, W, b):
    blk = 1000
    grid = (N // blk,)
    return pl.pallas_call(
        _finalize_body,
        grid=grid,
        in_specs=[
            pl.BlockSpec((blk, D), lambda i: (i, 0)),
            pl.BlockSpec((blk, D), lambda i: (i, 0)),
            pl.BlockSpec((D, D), lambda i: (0, 0)),
            pl.BlockSpec((1, D), lambda i: (0, 0)),
        ],
        out_specs=pl.BlockSpec((blk, D), lambda i: (i, 0)),
        out_shape=jax.ShapeDtypeStruct((N, D), jnp.float32),
    )(p0, p1, W, b)


@jax.jit
def kernel(x, adj_values, edge_index, W, b):
    pad = E_PAD - E
    src = jnp.concatenate(
        [edge_index[1].astype(jnp.int32), jnp.zeros((pad,), jnp.int32)])
    dst = jnp.concatenate(
        [edge_index[0].astype(jnp.int32), jnp.zeros((pad,), jnp.int32)])
    adj = jnp.concatenate(
        [adj_values.astype(jnp.float32), jnp.zeros((pad,), jnp.float32)])

    partials = _sc_aggregate(x, src, dst, adj)
    p0 = partials[:N]
    p1 = partials[N_PAD:N_PAD + N]
    return _tc_finalize(p0, p1, W, b.reshape(1, D))


# speedup vs baseline: 2.7870x; 1.0852x over previous
import functools

import jax
import jax.numpy as jnp
from jax import lax
from jax.experimental import pallas as pl
from jax.experimental.pallas import tpu as pltpu
from jax.experimental.pallas import tpu_sc as plsc

N = 10000
D = 128
E = 320000

NUM_CORES = 2
NUM_SUBCORES = 16
NUM_TILES = NUM_CORES * NUM_SUBCORES

CHUNK = 128
CHUNKS_PER_TILE = 80
CHUNKS_ALLOC = CHUNKS_PER_TILE + 1
E_PAD = NUM_TILES * CHUNKS_PER_TILE * CHUNK

N_PAD = 10240
ROWS_PER_SUBCORE = N_PAD // NUM_SUBCORES
ZERO_ROWS = 128

_SELU_ALPHA = 1.6732632423543772
_SELU_SCALE = 1.0507009873554805


def _sc_aggregate(x, src, dst, adj):
    mesh = plsc.VectorSubcoreMesh(core_axis_name="c", subcore_axis_name="s")

    @functools.partial(
        pl.kernel,
        mesh=mesh,
        out_type=jax.ShapeDtypeStruct((NUM_CORES * N_PAD, D), jnp.float32),
        scratch_types=[
            pltpu.VMEM((CHUNKS_ALLOC, CHUNK), jnp.int32),
            pltpu.VMEM((CHUNK,), jnp.int32),
            pltpu.VMEM((CHUNK,), jnp.int32),
            pltpu.VMEM((CHUNK,), jnp.float32),
            pltpu.VMEM((CHUNK,), jnp.float32),
            pltpu.VMEM((CHUNK, D), jnp.float32),
            pltpu.VMEM((CHUNK, D), jnp.float32),
            pltpu.VMEM_SHARED((N_PAD, D), jnp.float32),
            pltpu.SemaphoreType.DMA,
            pltpu.SemaphoreType.DMA,
            pltpu.SemaphoreType.DMA,
            pltpu.SemaphoreType.DMA,
        ],
    )
    def agg(x_hbm, src_hbm, dst_hbm, adj_hbm, out_hbm,
            srcv, dst0, dst1, adj0, adj1, rows0, rows1, acc,
            semg0, semg1, semd0, semd1):
        c = lax.axis_index("c")
        s = lax.axis_index("s")
        wid = c * NUM_SUBCORES + s

        pltpu.sync_copy(src_hbm.at[wid], srcv)

        def zero_row(r, carry):
            for g in range(D // 16):
                rows0[r, pl.ds(g * 16, 16)] = jnp.zeros((16,), jnp.float32)
            return carry

        lax.fori_loop(0, ZERO_ROWS, zero_row, 0)
        for k in range(ROWS_PER_SUBCORE // ZERO_ROWS):
            pltpu.sync_copy(
                rows0.at[pl.ds(0, ZERO_ROWS)],
                acc.at[pl.ds(s * ROWS_PER_SUBCORE + k * ZERO_ROWS, ZERO_ROWS)])
        plsc.subcore_barrier()

        def scale_rows(rows, adjb):
            def scale_group(g16, c2):
                r0 = g16 * 16
                avec = adjb[pl.ds(r0, 16)]
                for j in range(16):
                    a = avec[j]
                    for g in range(D // 16):
                        rows[r0 + j, pl.ds(g * 16, 16)] = (
                            rows[r0 + j, pl.ds(g * 16, 16)] * a)
                return c2

            lax.fori_loop(0, CHUNK // 16, scale_group, 0)

        cbase = wid * CHUNKS_ALLOC * CHUNK

        pltpu.async_copy(dst_hbm.at[pl.ds(cbase, CHUNK)], dst0, semd0)
        pltpu.async_copy(adj_hbm.at[pl.ds(cbase, CHUNK)], adj0, semd0)
        pltpu.async_copy(x_hbm.at[srcv.at[0]], rows0, semg0)

        def pipe_body(i, carry):
            a = 2 * i
            b = a + 1
            pltpu.async_copy(
                dst_hbm.at[pl.ds(cbase + b * CHUNK, CHUNK)], dst1, semd1)
            pltpu.async_copy(
                adj_hbm.at[pl.ds(cbase + b * CHUNK, CHUNK)], adj1, semd1)
            pltpu.async_copy(x_hbm.at[srcv.at[b]], rows1, semg1)
            pltpu.make_async_copy(x_hbm.at[srcv.at[a]], rows0, semg0).wait()
            pltpu.make_async_copy(
                dst_hbm.at[pl.ds(cbase + a * CHUNK, CHUNK)], dst0, semd0).wait()
            pltpu.make_async_copy(
                adj_hbm.at[pl.ds(cbase + a * CHUNK, CHUNK)], adj0, semd0).wait()
            scale_rows(rows0, adj0)
            pltpu.sync_copy(rows0, acc.at[dst0], add=True)
            pltpu.async_copy(
                dst_hbm.at[pl.ds(cbase + (a + 2) * CHUNK, CHUNK)], dst0, semd0)
            pltpu.async_copy(
                adj_hbm.at[pl.ds(cbase + (a + 2) * CHUNK, CHUNK)], adj0, semd0)
            pltpu.async_copy(x_hbm.at[srcv.at[a + 2]], rows0, semg0)
            pltpu.make_async_copy(x_hbm.at[srcv.at[b]], rows1, semg1).wait()
            pltpu.make_async_copy(
                dst_hbm.at[pl.ds(cbase + b * CHUNK, CHUNK)], dst1, semd1).wait()
            pltpu.make_async_copy(
                adj_hbm.at[pl.ds(cbase + b * CHUNK, CHUNK)], adj1, semd1).wait()
            scale_rows(rows1, adj1)
            pltpu.sync_copy(rows1, acc.at[dst1], add=True)
            return carry

        lax.fori_loop(0, CHUNKS_PER_TILE // 2, pipe_body, 0)
        pltpu.make_async_copy(
            x_hbm.at[srcv.at[CHUNKS_PER_TILE]], rows0, semg0).wait()
        pltpu.make_async_copy(
            dst_hbm.at[pl.ds(cbase + CHUNKS_PER_TILE * CHUNK, CHUNK)],
            dst0, semd0).wait()
        pltpu.make_async_copy(
            adj_hbm.at[pl.ds(cbase + CHUNKS_PER_TILE * CHUNK, CHUNK)],
            adj0, semd0).wait()
        plsc.subcore_barrier()

        for k in range(ROWS_PER_SUBCORE // ZERO_ROWS):
            r0 = s * ROWS_PER_SUBCORE + k * ZERO_ROWS
            pltpu.sync_copy(acc.at[pl.ds(r0, ZERO_ROWS)],
                            rows0.at[pl.ds(0, ZERO_ROWS)])
            pltpu.sync_copy(rows0.at[pl.ds(0, ZERO_ROWS)],
                            out_hbm.at[pl.ds(c * N_PAD + r0, ZERO_ROWS)])

    return agg(x, src, dst, adj)


def _finalize_body(p0_ref, p1_ref, w_ref, b_ref, o_ref):
    acc = p0_ref[...] + p1_ref[...]
    h = jnp.dot(acc, w_ref[...], preferred_element_type=jnp.float32)
    neg = _SELU_ALPHA * (jnp.exp(h) - 1.0)
    o_ref[...] = _SELU_SCALE * jnp.where(h > 0, h, neg) + b_ref[...]


def _tc_finalize(p0, p1, W, b):
    blk = 1000
    grid = (N // blk,)
    return pl.pallas_call(
        _finalize_body,
        grid=grid,
        in_specs=[
            pl.BlockSpec((blk, D), lambda i: (i, 0)),
            pl.BlockSpec((blk, D), lambda i: (i, 0)),
            pl.BlockSpec((D, D), lambda i: (0, 0)),
            pl.BlockSpec((1, D), lambda i: (0, 0)),
        ],
        out_specs=pl.BlockSpec((blk, D), lambda i: (i, 0)),
        out_shape=jax.ShapeDtypeStruct((N, D), jnp.float32),
    )(p0, p1, W, b)


@jax.jit
def kernel(x, adj_values, edge_index, W, b):
    pad = E_PAD - E
    shape3 = (NUM_TILES, CHUNKS_PER_TILE, CHUNK)
    src = jnp.concatenate(
        [edge_index[1].astype(jnp.int32), jnp.zeros((pad,), jnp.int32)])
    dst = jnp.concatenate(
        [edge_index[0].astype(jnp.int32), jnp.zeros((pad,), jnp.int32)])
    adj = jnp.concatenate(
        [adj_values.astype(jnp.float32), jnp.zeros((pad,), jnp.float32)])
    pad_chunk_i = jnp.zeros((NUM_TILES, 1, CHUNK), jnp.int32)
    pad_chunk_f = jnp.zeros((NUM_TILES, 1, CHUNK), jnp.float32)
    src = jnp.concatenate([src.reshape(shape3), pad_chunk_i], axis=1)
    dst = jnp.concatenate([dst.reshape(shape3), pad_chunk_i], axis=1).reshape(-1)
    adj = jnp.concatenate([adj.reshape(shape3), pad_chunk_f], axis=1).reshape(-1)

    partials = _sc_aggregate(x, src, dst, adj)
    p0 = partials[:N]
    p1 = partials[N_PAD:N_PAD + N]
    return _tc_finalize(p0, p1, W, b.reshape(1, D))
